# baseline (speedup 1.0000x reference)
import functools

import jax
import jax.numpy as jnp
from jax import lax
from jax.experimental import pallas as pl
from jax.experimental.pallas import tpu as pltpu
from jax.experimental.pallas import tpu_sc as plsc

EMBED_DIM = 64
COMMITMENT_COST = 0.25
ROW_BLOCK = 512

_NUM_CORES = 2
_NUM_SUBCORES = 16
_NUM_WORKERS = _NUM_CORES * _NUM_SUBCORES
_GATHER_CHUNK = 96


LANES = 128
CODE_BLOCK = 512


def _argmin_body(n_row_blocks, n_codes,
                 z_ref, embt_ref, idx_ref, loss_ref, esq_ref):
    i = pl.program_id(0)
    n_code_blocks = n_codes // CODE_BLOCK
    n_sub = CODE_BLOCK // LANES

    @pl.when(i == 0)
    def _():
        loss_ref[0, 0] = 0.0
        for j in range(n_code_blocks):
            embt_j = embt_ref[:, j * CODE_BLOCK:(j + 1) * CODE_BLOCK]
            esq_ref[0, j * CODE_BLOCK:(j + 1) * CODE_BLOCK] = (
                0.5 * jnp.sum(embt_j * embt_j, axis=0))

    z = z_ref[...]
    v = jnp.full((ROW_BLOCK, LANES), jnp.inf, jnp.float32)
    g = jnp.zeros((ROW_BLOCK, LANES), jnp.int32)
    for j in range(n_code_blocks):
        embt_j = embt_ref[:, j * CODE_BLOCK:(j + 1) * CODE_BLOCK]
        prod = lax.dot_general(
            z, embt_j, (((1,), (0,)), ((), ())),
            preferred_element_type=jnp.float32,
        )
        esq_j = esq_ref[0, j * CODE_BLOCK:(j + 1) * CODE_BLOCK]
        for c in range(n_sub):
            s = (esq_j[None, c * LANES:(c + 1) * LANES]
                 - prod[:, c * LANES:(c + 1) * LANES])
            better = s < v
            v = jnp.where(better, s, v)
            g = jnp.where(better, j * n_sub + c, g)

    lane = lax.broadcasted_iota(jnp.int32, (ROW_BLOCK, LANES), 1)
    vidx = g * LANES + lane
    minv = jnp.min(v, axis=1)
    idx = jnp.min(jnp.where(v == minv[:, None], vidx, n_codes), axis=1)
    idx_ref[0, 0, :] = idx
    z_sq = jnp.sum(z * z, axis=1)
    loss_ref[0, 0] += jnp.sum(z_sq + 2.0 * minv)

    @pl.when(i == n_row_blocks - 1)
    def _():
        loss_ref[0, 0] *= COMMITMENT_COST / (n_row_blocks * ROW_BLOCK * EMBED_DIM)


def _tc_argmin(flat_z, embedding):
    n_tokens = flat_z.shape[0]
    n_codes = embedding.shape[0]
    n_row_blocks = n_tokens // ROW_BLOCK
    idx3, loss = pl.pallas_call(
        functools.partial(_argmin_body, n_row_blocks, n_codes),
        grid=(n_row_blocks,),
        in_specs=[
            pl.BlockSpec((ROW_BLOCK, EMBED_DIM), lambda i: (i, 0)),
            pl.BlockSpec((EMBED_DIM, n_codes), lambda i: (0, 0)),
        ],
        out_specs=[
            pl.BlockSpec((1, 1, ROW_BLOCK), lambda i: (i, 0, 0)),
            pl.BlockSpec(memory_space=pltpu.SMEM),
        ],
        out_shape=[
            jax.ShapeDtypeStruct((n_row_blocks, 1, ROW_BLOCK), jnp.int32),
            jax.ShapeDtypeStruct((1, 1), jnp.float32),
        ],
        scratch_shapes=[
            pltpu.VMEM((1, n_codes), jnp.float32),
        ],
    )(flat_z, embedding.T)
    return idx3.reshape(n_tokens), loss[0, 0]


def _sc_gather(indices, table_padded):
    n_tokens = indices.shape[0]
    width = table_padded.shape[1]
    per_worker = n_tokens // _NUM_WORKERS
    n_chunks = per_worker // _GATHER_CHUNK
    mesh = plsc.VectorSubcoreMesh(core_axis_name="c", subcore_axis_name="s")

    @functools.partial(
        pl.kernel,
        mesh=mesh,
        out_type=jax.ShapeDtypeStruct((n_tokens, width), jnp.float32),
        scratch_types=[
            pltpu.VMEM((per_worker,), jnp.int32),
            pltpu.VMEM((per_worker, width), jnp.float32),
            pltpu.SemaphoreType.DMA,
        ],
    )
    def gather(idx_hbm, table_hbm, out_hbm, idx_v, rows_v, sem):
        wid = lax.axis_index("s") * _NUM_CORES + lax.axis_index("c")
        base = wid * per_worker
        pltpu.sync_copy(idx_hbm.at[pl.ds(base, per_worker)], idx_v)
        copies = [
            pltpu.async_copy(
                table_hbm.at[idx_v.at[pl.ds(j * _GATHER_CHUNK, _GATHER_CHUNK)]],
                rows_v.at[pl.ds(j * _GATHER_CHUNK, _GATHER_CHUNK)],
                sem)
            for j in range(n_chunks)
        ]
        for c in copies:
            c.wait()
        pltpu.sync_copy(rows_v, out_hbm.at[pl.ds(base, per_worker)])

    return gather(indices, table_padded)


def kernel(z, embedding):
    flat_z = z.reshape(-1, EMBED_DIM)
    indices, loss = _tc_argmin(flat_z, embedding)
    table_padded = jnp.pad(embedding, ((0, 0), (0, 128 - EMBED_DIM)))
    z_q = _sc_gather(indices, table_padded)[:, :EMBED_DIM]
    return z_q.reshape(z.shape), loss, indices
